# baseline (speedup 1.0000x reference)
import jax
import jax.numpy as jnp
from jax import lax
from jax.experimental import pallas as pl
from jax.experimental.pallas import tpu as pltpu
from jax.experimental.pallas import tpu_sc as plsc

ENC_MAX = 5
B = 16384
NC, NS = 2, 16
NW = NC * NS
BPW = B // NW
CHUNK = 32
NCH = BPW // CHUNK
NT = 4
PAIR = 26 * 26
CCN = 6 * 256
ROWS = 3 * PAIR + CCN
ROWS_PAD = 3568
D = 128


def _maxnorm(t):
    n = jnp.sqrt(jnp.sum(t * t, axis=-1, keepdims=True))
    scale = jnp.minimum(1.0, 1.0 / jnp.maximum(n, 1e-7))
    return t * scale


def _prep_body(packed_ref, ohacl_ref, wfeat_ref, spc8_ref,
               s0_ref, s1_ref, s2_ref, s3_ref,
               s4_ref, s5_ref, cc_ref, ar_ref,
               btable_ref, cidx_ref, spc_ref, vstr_ref):
    f32 = jnp.float32
    packed = packed_ref[...]
    stat = _maxnorm(packed[0:26, 0:32])
    wfeat = wfeat_ref[...]

    proj = [lax.dot_general(stat, wfeat[:, 32 * k:32 * k + 32],
                            (((1,), (1,)), ((), ())),
                            preferred_element_type=f32) for k in range(6)]

    r1 = lax.broadcasted_iota(jnp.int32, (PAIR, 26), 0)
    c1 = lax.broadcasted_iota(jnp.int32, (PAIR, 26), 1)
    oh_i = (r1 // 26 == c1).astype(f32)
    oh_j = (r1 % 26 == c1).astype(f32)
    pairs = [lax.dot_general(oh_i, proj[2 * p], (((1,), (0,)), ((), ())),
                             preferred_element_type=f32)
             + lax.dot_general(oh_j, proj[2 * p + 1], (((1,), (0,)), ((), ())),
                               preferred_element_type=f32)
             for p in range(3)]

    kind = packed[26:32, 0:32]
    const = packed[62:63, :]
    for k in range(6):
        const = const + lax.dot_general(
            kind[k:k + 1, :], wfeat[:, 32 * k:32 * k + 32],
            (((1,), (1,)), ((), ())), preferred_element_type=f32)

    enc = _maxnorm(packed[32:38, 0:8])
    erow = lax.broadcasted_iota(jnp.int32, (6, 8), 0)
    enc = jnp.where(erow == ENC_MAX, 0.0, enc)
    ac = _maxnorm(packed[38:62, 0:8])
    acfull = jnp.dot(ohacl_ref[...][:, 0:24], ac,
                     preferred_element_type=f32)
    ep = lax.dot_general(enc, wfeat[:, 192:200], (((1,), (1,)), ((), ())),
                         preferred_element_type=f32)
    afp = lax.dot_general(acfull, wfeat[:, 200:208], (((1,), (1,)), ((), ())),
                          preferred_element_type=f32)
    r2 = lax.broadcasted_iota(jnp.int32, (CCN, 6), 0)
    oh_c = (r2 // 256 == lax.broadcasted_iota(jnp.int32, (CCN, 6), 1)
            ).astype(f32)
    r3 = lax.broadcasted_iota(jnp.int32, (CCN, 256), 0)
    oh_a = (r3 % 256 == lax.broadcasted_iota(jnp.int32, (CCN, 256), 1)
            ).astype(f32)
    ccac = (lax.dot_general(oh_c, ep, (((1,), (0,)), ((), ())),
                            preferred_element_type=f32)
            + lax.dot_general(oh_a, afp, (((1,), (0,)), ((), ())),
                              preferred_element_type=f32)
            + const)

    btable_ref[...] = jnp.concatenate(
        pairs + [ccac, jnp.zeros((ROWS_PAD - ROWS, D), f32)], axis=0)

    i01 = s0_ref[...] * 26 + s1_ref[...]
    i23 = s2_ref[...] * 26 + s3_ref[...] + PAIR
    i45 = s4_ref[...] * 26 + s5_ref[...] + 2 * PAIR
    icc = cc_ref[...] * 256 + ar_ref[...] + 3 * PAIR
    cidx_ref[...] = jnp.concatenate([i01, i23, i45, icc], axis=0)

    rep = (lax.broadcasted_iota(jnp.int32, (8, 128), 1) // 16
           == lax.broadcasted_iota(jnp.int32, (8, 128), 0)).astype(f32)
    spc_ref[...] = jnp.dot(spc8_ref[...], rep, preferred_element_type=f32)

    vstr = lax.dot_general(packed[63:64, 0:32], wfeat[:, 0:32],
                           (((1,), (1,)), ((), ())),
                           preferred_element_type=f32) / 99.0
    vstr_ref[...] = jnp.broadcast_to(vstr, (8, D))


def _prep(args):
    return pl.pallas_call(
        _prep_body,
        out_shape=(
            jax.ShapeDtypeStruct((ROWS_PAD, D), jnp.float32),
            jax.ShapeDtypeStruct((NT * 128, 128), jnp.int32),
            jax.ShapeDtypeStruct((B // 8, 128), jnp.float32),
            jax.ShapeDtypeStruct((8, D), jnp.float32),
        ),
    )(*args)


def _gather_body(btable_hbm, cidx_hbm, spc_hbm, vstr_hbm, out_hbm,
                 idx_v, bufs_v, obuf_v, spc_v, vstr_v,
                 gsem0, gsem1, wsem0, wsem1):
    gsems = [gsem0, gsem1]
    wsems = [wsem0, wsem1]
    wid = lax.axis_index("s") * NC + lax.axis_index("c")
    base = wid * BPW
    rpw = BPW // 128
    for t in range(NT):
        pltpu.sync_copy(cidx_hbm.at[pl.ds(t * 128 + wid * rpw, rpw), :],
                        idx_v.at[t])
    pltpu.sync_copy(spc_hbm.at[pl.ds(wid * (BPW // 8), BPW // 8), :], spc_v)
    pltpu.sync_copy(vstr_hbm.at[0], vstr_v)
    vstr_regs = [vstr_v[pl.ds(16 * c, 16)] for c in range(D // 16)]

    cpr = 128 // CHUNK

    def fire(j):
        s = j % 2
        return [pltpu.async_copy(
            btable_hbm.at[idx_v.at[t, j // cpr,
                                   pl.ds((j % cpr) * CHUNK, CHUNK)]],
            bufs_v.at[s, t], gsems[s]) for t in range(NT)]

    gcps = fire(0)
    wcps = [None, None]
    for j in range(NCH):
        s = j % 2
        nxt = fire(j + 1) if j + 1 < NCH else []
        for cp in gcps:
            cp.wait()
        gcps = nxt
        if wcps[s] is not None:
            wcps[s].wait()

        def grp_body(p, _, j=j, s=s):
            for q in range(8):
                spc = spc_v[j * (CHUNK // 8) + p, pl.ds(16 * q, 16)]
                r = p * 8 + q
                for c in range(D // 16):
                    sl = pl.ds(16 * c, 16)
                    obuf_v[s, r, sl] = (bufs_v[s, 0, r, sl]
                                        + bufs_v[s, 1, r, sl]
                                        + bufs_v[s, 2, r, sl]
                                        + bufs_v[s, 3, r, sl]
                                        + spc * vstr_regs[c])
            return _

        lax.fori_loop(0, CHUNK // 8, grp_body, 0)
        wcps[s] = pltpu.async_copy(
            obuf_v.at[s], out_hbm.at[pl.ds(base + j * CHUNK, CHUNK), :],
            wsems[s])
    for cp in wcps:
        if cp is not None:
            cp.wait()


def _gather(btable, cidx, spc, vstr):
    mesh = plsc.VectorSubcoreMesh(core_axis_name="c", subcore_axis_name="s")
    f = pl.kernel(
        _gather_body,
        out_type=jax.ShapeDtypeStruct((B, D), jnp.float32),
        mesh=mesh,
        scratch_types=[
            pltpu.VMEM((NT, BPW // 128, 128), jnp.int32),
            pltpu.VMEM((2, NT, CHUNK, D), jnp.float32),
            pltpu.VMEM((2, CHUNK, D), jnp.float32),
            pltpu.VMEM((BPW // 8, 128), jnp.float32),
            pltpu.VMEM((D,), jnp.float32),
            pltpu.SemaphoreType.DMA,
            pltpu.SemaphoreType.DMA,
            pltpu.SemaphoreType.DMA,
            pltpu.SemaphoreType.DMA,
        ],
    )
    return f(btable, cidx, spc, vstr)


def kernel(str_, dex, con, int_, wis, cha, strength_percentage, armor_class,
           carrying_capacity, stat_table, kind_table, W_str, enc_table,
           ac_lookup, ac_table, W_feat, b_feat):
    i32 = jnp.int32

    def blk(x):
        return x.astype(i32).reshape(128, 128)

    packed = jnp.concatenate([
        jnp.pad(stat_table, ((0, 0), (0, 96))),
        jnp.pad(kind_table, ((0, 0), (0, 96))),
        jnp.pad(enc_table, ((0, 0), (0, 120))),
        jnp.pad(ac_table, ((0, 0), (0, 120))),
        b_feat.reshape(1, 128),
        jnp.pad(W_str.reshape(1, 32), ((0, 0), (0, 96))),
    ], axis=0)
    ohacl = (ac_lookup.astype(i32)[:, None]
             == jnp.arange(128, dtype=i32)[None, :]).astype(jnp.float32)
    wfeat_p = jnp.pad(W_feat, ((0, 0), (0, 48)))

    btable, cidx, spc, vstr = _prep((
        packed, ohacl, wfeat_p, strength_percentage.reshape(B // 8, 8),
        blk(str_), blk(dex), blk(con), blk(int_), blk(wis), blk(cha),
        blk(carrying_capacity), blk(armor_class),
    ))
    return _gather(btable, cidx, spc, vstr)


# speedup vs baseline: 1.1569x; 1.1569x over previous
import jax
import jax.numpy as jnp
from jax import lax
from jax.experimental import pallas as pl
from jax.experimental.pallas import tpu as pltpu
from jax.experimental.pallas import tpu_sc as plsc

ENC_MAX = 5
B = 16384
NC, NS = 2, 16
NW = NC * NS
BPW = B // NW
CHUNK = 64
NCH = BPW // CHUNK
NT = 4
PAIR = 26 * 26
CCN = 6 * 256
ROWS = 3 * PAIR + CCN
ROWS_PAD = 3568
D = 128


def _maxnorm(t):
    n = jnp.sqrt(jnp.sum(t * t, axis=-1, keepdims=True))
    scale = jnp.minimum(1.0, 1.0 / jnp.maximum(n, 1e-7))
    return t * scale


def _prep_body(stat_ref, kind_ref, wstr_ref, enc_ref, acl_ref, ac_ref,
               wfeat_ref, bfeat_ref, s0_ref, s1_ref, s2_ref, s3_ref,
               s4_ref, s5_ref, cc_ref, ar_ref, spc8_ref,
               btable_ref, cidx_ref, spc_ref, vstr_ref):
    f32 = jnp.float32
    stat = _maxnorm(stat_ref[...])
    wfeat = wfeat_ref[...]

    proj = [lax.dot_general(stat, wfeat[:, 32 * k:32 * k + 32],
                            (((1,), (1,)), ((), ())),
                            preferred_element_type=f32) for k in range(6)]

    r1 = lax.broadcasted_iota(jnp.int32, (PAIR, 26), 0)
    c1 = lax.broadcasted_iota(jnp.int32, (PAIR, 26), 1)
    oh_i = (r1 // 26 == c1).astype(f32)
    oh_j = (r1 % 26 == c1).astype(f32)
    pairs = [lax.dot_general(oh_i, proj[2 * p], (((1,), (0,)), ((), ())),
                             preferred_element_type=f32)
             + lax.dot_general(oh_j, proj[2 * p + 1], (((1,), (0,)), ((), ())),
                               preferred_element_type=f32)
             for p in range(3)]

    kind = kind_ref[...]
    const = bfeat_ref[...]
    for k in range(6):
        const = const + lax.dot_general(
            kind[k:k + 1, :], wfeat[:, 32 * k:32 * k + 32],
            (((1,), (1,)), ((), ())), preferred_element_type=f32)

    enc = _maxnorm(enc_ref[...])
    erow = lax.broadcasted_iota(jnp.int32, (6, 8), 0)
    enc = jnp.where(erow == ENC_MAX, 0.0, enc)
    ac = _maxnorm(ac_ref[...])
    oh_acl = (acl_ref[...] == lax.broadcasted_iota(jnp.int32, (256, 24), 1)
              ).astype(f32)
    acfull = jnp.dot(oh_acl, ac, preferred_element_type=f32)
    ep = lax.dot_general(enc, wfeat[:, 192:200], (((1,), (1,)), ((), ())),
                         preferred_element_type=f32)
    afp = lax.dot_general(acfull, wfeat[:, 200:208], (((1,), (1,)), ((), ())),
                          preferred_element_type=f32)
    r2 = lax.broadcasted_iota(jnp.int32, (CCN, 6), 0)
    oh_c = (r2 // 256 == lax.broadcasted_iota(jnp.int32, (CCN, 6), 1)
            ).astype(f32)
    r3 = lax.broadcasted_iota(jnp.int32, (CCN, 256), 0)
    oh_a = (r3 % 256 == lax.broadcasted_iota(jnp.int32, (CCN, 256), 1)
            ).astype(f32)
    ccac = (lax.dot_general(oh_c, ep, (((1,), (0,)), ((), ())),
                            preferred_element_type=f32)
            + lax.dot_general(oh_a, afp, (((1,), (0,)), ((), ())),
                              preferred_element_type=f32)
            + const)

    btable_ref[...] = jnp.concatenate(
        pairs + [ccac, jnp.zeros((ROWS_PAD - ROWS, D), f32)], axis=0)

    i01 = s0_ref[...] * 26 + s1_ref[...]
    i23 = s2_ref[...] * 26 + s3_ref[...] + PAIR
    i45 = s4_ref[...] * 26 + s5_ref[...] + 2 * PAIR
    icc = cc_ref[...] * 256 + ar_ref[...] + 3 * PAIR
    cidx_ref[...] = jnp.concatenate([i01, i23, i45, icc], axis=0)

    rep = (lax.broadcasted_iota(jnp.int32, (8, 128), 1) // 16
           == lax.broadcasted_iota(jnp.int32, (8, 128), 0)).astype(f32)
    spc_ref[...] = jnp.dot(spc8_ref[...], rep, preferred_element_type=f32)

    vstr = lax.dot_general(wstr_ref[...], wfeat[:, 0:32],
                           (((0,), (1,)), ((), ())),
                           preferred_element_type=f32) / 99.0
    vstr_ref[...] = jnp.broadcast_to(vstr, (8, D))


def _prep(args):
    return pl.pallas_call(
        _prep_body,
        out_shape=(
            jax.ShapeDtypeStruct((ROWS_PAD, D), jnp.float32),
            jax.ShapeDtypeStruct((NT * 128, 128), jnp.int32),
            jax.ShapeDtypeStruct((B // 8, 128), jnp.float32),
            jax.ShapeDtypeStruct((8, D), jnp.float32),
        ),
    )(*args)


def _gather_body(btable_hbm, cidx_hbm, spc_hbm, vstr_hbm, out_hbm,
                 idx_v, bufs_v, spc_v, vstr_v,
                 gsem0, gsem1, wsem0, wsem1):
    gsems = [gsem0, gsem1]
    wsems = [wsem0, wsem1]
    wid = lax.axis_index("s") * NC + lax.axis_index("c")
    base = wid * BPW
    rpw = BPW // 128
    for t in range(NT):
        pltpu.sync_copy(cidx_hbm.at[pl.ds(t * 128 + wid * rpw, rpw), :],
                        idx_v.at[t])
    pltpu.sync_copy(spc_hbm.at[pl.ds(wid * (BPW // 8), BPW // 8), :], spc_v)
    pltpu.sync_copy(vstr_hbm.at[0], vstr_v)
    vstr_regs = [vstr_v[pl.ds(16 * c, 16)] for c in range(D // 16)]

    cpr = 128 // CHUNK

    def fire(j):
        s = j % 2
        return [pltpu.async_copy(
            btable_hbm.at[idx_v.at[t, j // cpr,
                                   pl.ds((j % cpr) * CHUNK, CHUNK)]],
            bufs_v.at[s, t], gsems[s]) for t in range(NT)]

    gcps = fire(0)
    wcps = [None, None]
    for j in range(NCH):
        s = j % 2
        nxt = []
        if j + 1 < NCH:
            if wcps[1 - s] is not None:
                wcps[1 - s].wait()
                wcps[1 - s] = None
            nxt = fire(j + 1)
        for cp in gcps:
            cp.wait()
        gcps = nxt

        def grp_body(p, _, j=j, s=s):
            for q in range(8):
                spc = spc_v[j * (CHUNK // 8) + p, pl.ds(16 * q, 16)]
                r = p * 8 + q
                for c in range(D // 16):
                    sl = pl.ds(16 * c, 16)
                    bufs_v[s, 0, r, sl] = (bufs_v[s, 0, r, sl]
                                           + bufs_v[s, 1, r, sl]
                                           + bufs_v[s, 2, r, sl]
                                           + bufs_v[s, 3, r, sl]
                                           + spc * vstr_regs[c])
            return _

        lax.fori_loop(0, CHUNK // 8, grp_body, 0)
        wcps[s] = pltpu.async_copy(
            bufs_v.at[s, 0], out_hbm.at[pl.ds(base + j * CHUNK, CHUNK), :],
            wsems[s])
    for cp in wcps:
        if cp is not None:
            cp.wait()


def _gather(btable, cidx, spc, vstr):
    mesh = plsc.VectorSubcoreMesh(core_axis_name="c", subcore_axis_name="s")
    f = pl.kernel(
        _gather_body,
        out_type=jax.ShapeDtypeStruct((B, D), jnp.float32),
        mesh=mesh,
        scratch_types=[
            pltpu.VMEM((NT, BPW // 128, 128), jnp.int32),
            pltpu.VMEM((2, NT, CHUNK, D), jnp.float32),
            pltpu.VMEM((BPW // 8, 128), jnp.float32),
            pltpu.VMEM((D,), jnp.float32),
            pltpu.SemaphoreType.DMA,
            pltpu.SemaphoreType.DMA,
            pltpu.SemaphoreType.DMA,
            pltpu.SemaphoreType.DMA,
        ],
    )
    return f(btable, cidx, spc, vstr)


def kernel(str_, dex, con, int_, wis, cha, strength_percentage, armor_class,
           carrying_capacity, stat_table, kind_table, W_str, enc_table,
           ac_lookup, ac_table, W_feat, b_feat):
    i32 = jnp.int32

    def blk(x):
        return x.astype(i32).reshape(128, 128)

    btable, cidx, spc, vstr = _prep((
        stat_table, kind_table, W_str, enc_table,
        ac_lookup.astype(i32).reshape(256, 1), ac_table,
        W_feat, b_feat.reshape(1, D),
        blk(str_), blk(dex), blk(con), blk(int_), blk(wis), blk(cha),
        blk(carrying_capacity), blk(armor_class),
        strength_percentage.reshape(B // 8, 8),
    ))
    return _gather(btable, cidx, spc, vstr)
